# baseline (speedup 1.0000x reference)
import functools

import jax
import jax.numpy as jnp
from jax import lax
from jax.experimental import pallas as pl
from jax.experimental.pallas import tpu as pltpu
from jax.experimental.pallas import tpu_sc as plsc

_TEMP = 0.5
_EPS = 1e-3

_SC_PARAMS = dict(
    compiler_params=pltpu.CompilerParams(
        use_tc_tiling_on_sc=False, needs_layout_passes=False),
)


def _make_sc_emb_gather(n_rows, d, n_idx, nw, ch):
    per_w = ch * 128
    group = 13
    mesh = plsc.VectorSubcoreMesh(core_axis_name="c", subcore_axis_name="s")
    info = plsc.get_sparse_core_info()
    nc = info.num_cores

    @functools.partial(
        pl.kernel,
        mesh=mesh,
        out_type=[jax.ShapeDtypeStruct((n_idx, d), jnp.float32)],
        scratch_types=[
            pltpu.VMEM((ch, 128), jnp.int32),
            pltpu.VMEM((per_w, d), jnp.float32),
            pltpu.SemaphoreType.DMA,
        ],
        **_SC_PARAMS,
    )
    def sc_emb_gather(emb_hbm, idx_hbm, rows_out, idx_v, rows_v, sem_e):
        wid = lax.axis_index("s") * nc + lax.axis_index("c")
        pltpu.sync_copy(idx_hbm.at[wid], idx_v)

        def body(j0):
            copies = [
                pltpu.async_copy(
                    emb_hbm.at[idx_v.at[j0 + k]],
                    rows_v.at[pl.ds((j0 + k) * 128, 128)], sem_e)
                for k in range(group)
            ]
            for c in copies:
                c.wait()

        pl.loop(0, ch, step=group)(body)
        pltpu.sync_copy(rows_v, rows_out.at[pl.ds(wid * per_w, per_w)])

    return sc_emb_gather


def _make_sc_bias_gather(n_rows16, d, n_idx, nw, ch):
    per_w = ch * 128
    group = 13
    mesh = plsc.VectorSubcoreMesh(core_axis_name="c", subcore_axis_name="s")
    info = plsc.get_sparse_core_info()
    nc = info.num_cores

    @functools.partial(
        pl.kernel,
        mesh=mesh,
        out_type=[jax.ShapeDtypeStruct((n_idx,), jnp.float32)],
        scratch_types=[
            pltpu.VMEM((ch, 128), jnp.int32),
            pltpu.VMEM((per_w,), jnp.int32),
            pltpu.VMEM((group * 128, d), jnp.float32),
            pltpu.VMEM((per_w,), jnp.float32),
            pltpu.SemaphoreType.DMA,
        ],
        **_SC_PARAMS,
    )
    def sc_bias_gather(bias16_hbm, bidx_hbm, lidx_hbm, bias_out,
                       bidx_v, lidx_v, brows_v, bias_v, sem_b):
        wid = lax.axis_index("s") * nc + lax.axis_index("c")
        pltpu.sync_copy(bidx_hbm.at[wid], bidx_v)
        pltpu.sync_copy(lidx_hbm.at[wid], lidx_v)

        def body(j0):
            copies = [
                pltpu.async_copy(
                    bias16_hbm.at[bidx_v.at[j0 + k]],
                    brows_v.at[pl.ds(k * 128, 128)], sem_b)
                for k in range(group)
            ]
            for c in copies:
                c.wait()
            iota = lax.iota(jnp.int32, 16)
            zeros = iota * 0
            for k in range(group):
                for t in range(8):
                    base = k * 128 + t * 16
                    lanes = lidx_v[pl.ds(j0 * 128 + base, 16)]
                    vals = plsc.load_gather(
                        brows_v, [iota + base, lanes])
                    bias_v[pl.ds(j0 * 128 + base, 16)] = vals

        pl.loop(0, ch, step=group)(body)
        pltpu.sync_copy(bias_v, bias_out.at[pl.ds(wid * per_w, per_w)])

    return sc_bias_gather


def _stats_body(emb_ref, n0_ref, n1_ref, scale_ref, shift_ref):
    x = emb_ref[...]
    b = x.shape[0]
    mean = jnp.sum(x, axis=0, keepdims=True) * (1.0 / b)
    var = jnp.sum(x * x, axis=0, keepdims=True) * (1.0 / b) - mean * mean
    gate_logit = (n1_ref[...] - n0_ref[...]) * (1.0 / _TEMP)
    c = 1.0 / (1.0 + jnp.exp(-gate_logit))
    s = c * lax.rsqrt(var + _EPS)
    scale_ref[...] = s
    shift_ref[...] = -mean * s


def _mlp_body(emb_ref, bias_ref, scale_ref, shift_ref,
              w1_ref, b1_ref, w2_ref, b2_ref, w3_ref, b3_ref, w4_ref, b4_ref,
              out_ref):
    x = emb_ref[...] * scale_ref[...] + shift_ref[...]
    h = jnp.dot(x, w1_ref[...], preferred_element_type=jnp.float32)
    h = jnp.maximum(h + b1_ref[...], 0.0)
    h = jnp.dot(h, w2_ref[...], preferred_element_type=jnp.float32)
    h = jnp.maximum(h + b2_ref[...], 0.0)
    h = jnp.dot(h, w3_ref[...], preferred_element_type=jnp.float32)
    h = jnp.maximum(h + b3_ref[...], 0.0)
    o = jnp.dot(h, w4_ref[...], preferred_element_type=jnp.float32)
    logit = o + b4_ref[...] + jnp.sum(bias_ref[...], axis=1, keepdims=True)
    out_ref[...] = 1.0 / (1.0 + jnp.exp(-logit))


def kernel(inputs, emb_table, bias_table, nas_logits,
           W1, b1, W2, b2, W3, b3, W4, b4):
    B, F = inputs.shape
    _, V, D = emb_table.shape
    NW = 32
    per_w = (B * F) // NW
    CH = per_w // 128

    idx = inputs.astype(jnp.int32) + (jnp.arange(F, dtype=jnp.int32) * V)[None, :]
    idx3 = idx.reshape(NW, CH, 128)
    bidx3 = idx3
    lidx2 = jnp.zeros((NW, CH * 128), jnp.int32)
    emb_flat = emb_table.reshape(F * V, D)
    bias16 = jnp.broadcast_to(bias_table, (F, V, D)).reshape(F * V, D)

    rows, = _make_sc_emb_gather(F * V, D, B * F, NW, CH)(emb_flat, idx3)
    bias_g, = _make_sc_bias_gather(F * V, D, B * F, NW, CH)(
        bias16, bidx3, lidx2)
    embs = rows.reshape(B, F * D)
    biases = bias_g.reshape(B, F)

    n0 = jnp.repeat(nas_logits[:, 0], D).reshape(1, F * D)
    n1 = jnp.repeat(nas_logits[:, 1], D).reshape(1, F * D)

    scale, shift = pl.pallas_call(
        _stats_body,
        out_shape=[jax.ShapeDtypeStruct((1, F * D), jnp.float32)] * 2,
    )(embs, n0, n1)

    BM = 512
    NB = B // BM

    def cmap(i):
        return (0, 0)

    out = pl.pallas_call(
        _mlp_body,
        grid=(NB,),
        in_specs=[
            pl.BlockSpec((BM, F * D), lambda i: (i, 0)),
            pl.BlockSpec((BM, F), lambda i: (i, 0)),
            pl.BlockSpec((1, F * D), cmap),
            pl.BlockSpec((1, F * D), cmap),
            pl.BlockSpec(W1.shape, cmap),
            pl.BlockSpec((1, W1.shape[1]), cmap),
            pl.BlockSpec(W2.shape, cmap),
            pl.BlockSpec((1, W2.shape[1]), cmap),
            pl.BlockSpec(W3.shape, cmap),
            pl.BlockSpec((1, W3.shape[1]), cmap),
            pl.BlockSpec(W4.shape, cmap),
            pl.BlockSpec((1, W4.shape[1]), cmap),
        ],
        out_specs=pl.BlockSpec((BM, 1), lambda i: (i, 0)),
        out_shape=jax.ShapeDtypeStruct((B, 1), jnp.float32),
    )(embs, biases, scale, shift,
      W1, b1.reshape(1, -1), W2, b2.reshape(1, -1),
      W3, b3.reshape(1, -1), W4, b4.reshape(1, -1))

    return out.reshape(B)


# speedup vs baseline: 2.4568x; 2.4568x over previous
import functools

import jax
import jax.numpy as jnp
from jax import lax
from jax.experimental import pallas as pl
from jax.experimental.pallas import tpu as pltpu
from jax.experimental.pallas import tpu_sc as plsc

_TEMP = 0.5
_EPS = 1e-3

_SC_PARAMS = dict(
    compiler_params=pltpu.CompilerParams(
        use_tc_tiling_on_sc=False, needs_layout_passes=False),
)


def _make_sc_emb_gather(n_rows, d, n_idx, nw, ch):
    per_w = ch * 128
    group = 13
    mesh = plsc.VectorSubcoreMesh(core_axis_name="c", subcore_axis_name="s")
    info = plsc.get_sparse_core_info()
    nc = info.num_cores

    @functools.partial(
        pl.kernel,
        mesh=mesh,
        out_type=[jax.ShapeDtypeStruct((n_idx, d), jnp.float32)],
        scratch_types=[
            pltpu.VMEM((ch, 128), jnp.int32),
            pltpu.VMEM((per_w, d), jnp.float32),
            pltpu.SemaphoreType.DMA,
        ],
        **_SC_PARAMS,
    )
    def sc_emb_gather(emb_hbm, idx_hbm, rows_out, idx_v, rows_v, sem_e):
        wid = lax.axis_index("s") * nc + lax.axis_index("c")
        pltpu.sync_copy(idx_hbm.at[wid], idx_v)

        def body(j0):
            copies = [
                pltpu.async_copy(
                    emb_hbm.at[idx_v.at[j0 + k]],
                    rows_v.at[pl.ds((j0 + k) * 128, 128)], sem_e)
                for k in range(group)
            ]
            for c in copies:
                c.wait()

        pl.loop(0, ch, step=group)(body)
        pltpu.sync_copy(rows_v, rows_out.at[pl.ds(wid * per_w, per_w)])

    return sc_emb_gather


def _make_sc_bias_gather(n_rows16, d, n_idx, nw, ch):
    per_w = ch * 128
    group = 13
    mesh = plsc.VectorSubcoreMesh(core_axis_name="c", subcore_axis_name="s")
    info = plsc.get_sparse_core_info()
    nc = info.num_cores

    @functools.partial(
        pl.kernel,
        mesh=mesh,
        out_type=[jax.ShapeDtypeStruct((n_idx,), jnp.float32)],
        scratch_types=[
            pltpu.VMEM((ch, 128), jnp.int32),
            pltpu.VMEM((per_w,), jnp.int32),
            pltpu.VMEM((group * 128, 1, d), jnp.float32),
            pltpu.VMEM((per_w,), jnp.float32),
            pltpu.SemaphoreType.DMA,
        ],
        **_SC_PARAMS,
    )
    def sc_bias_gather(bias16_hbm, bidx_hbm, lidx_hbm, bias_out,
                       bidx_v, lidx_v, brows_v, bias_v, sem_b):
        wid = lax.axis_index("s") * nc + lax.axis_index("c")
        pltpu.sync_copy(bidx_hbm.at[wid], bidx_v)
        pltpu.sync_copy(lidx_hbm.at[wid], lidx_v)

        def body(j0):
            copies = [
                pltpu.async_copy(
                    bias16_hbm.at[bidx_v.at[j0 + k]],
                    brows_v.at[pl.ds(k * 128, 128)], sem_b)
                for k in range(group)
            ]
            for c in copies:
                c.wait()
            iota = lax.iota(jnp.int32, 16)
            zeros = iota * 0
            for k in range(group):
                for t in range(8):
                    base = k * 128 + t * 16
                    lanes = lidx_v[pl.ds(j0 * 128 + base, 16)]
                    vals = plsc.load_gather(
                        brows_v, [iota + base, zeros, lanes])
                    bias_v[pl.ds(j0 * 128 + base, 16)] = vals

        pl.loop(0, ch, step=group)(body)
        pltpu.sync_copy(bias_v, bias_out.at[pl.ds(wid * per_w, per_w)])

    return sc_bias_gather


def _stats_body(emb_ref, n0_ref, n1_ref, scale_ref, shift_ref):
    x = emb_ref[...]
    b = x.shape[0]
    mean = jnp.sum(x, axis=0, keepdims=True) * (1.0 / b)
    var = jnp.sum(x * x, axis=0, keepdims=True) * (1.0 / b) - mean * mean
    gate_logit = (n1_ref[...] - n0_ref[...]) * (1.0 / _TEMP)
    c = 1.0 / (1.0 + jnp.exp(-gate_logit))
    s = c * lax.rsqrt(var + _EPS)
    scale_ref[...] = s
    shift_ref[...] = -mean * s


def _mlp_body(emb_ref, bias_ref, scale_ref, shift_ref,
              w1_ref, b1_ref, w2_ref, b2_ref, w3_ref, b3_ref, w4_ref, b4_ref,
              out_ref):
    x = emb_ref[...] * scale_ref[...] + shift_ref[...]
    h = jnp.dot(x, w1_ref[...], preferred_element_type=jnp.float32)
    h = jnp.maximum(h + b1_ref[...], 0.0)
    h = jnp.dot(h, w2_ref[...], preferred_element_type=jnp.float32)
    h = jnp.maximum(h + b2_ref[...], 0.0)
    h = jnp.dot(h, w3_ref[...], preferred_element_type=jnp.float32)
    h = jnp.maximum(h + b3_ref[...], 0.0)
    o = jnp.dot(h, w4_ref[...], preferred_element_type=jnp.float32)
    logit = o + b4_ref[...] + jnp.sum(bias_ref[...], axis=1, keepdims=True)
    out_ref[...] = 1.0 / (1.0 + jnp.exp(-logit))


def kernel(inputs, emb_table, bias_table, nas_logits,
           W1, b1, W2, b2, W3, b3, W4, b4):
    B, F = inputs.shape
    _, V, D = emb_table.shape
    NW = 32
    per_w = (B * F) // NW
    CH = per_w // 128

    idx = inputs.astype(jnp.int32) + (jnp.arange(F, dtype=jnp.int32) * V)[None, :]
    idx3 = idx.reshape(NW, CH, 128)
    bidx3 = (idx // D).reshape(NW, CH, 128)
    lidx2 = (idx % D).reshape(NW, CH * 128)
    emb_flat = emb_table.reshape(F * V, D)
    bias16 = bias_table.reshape((F * V) // D, 1, D)

    rows, = _make_sc_emb_gather(F * V, D, B * F, NW, CH)(emb_flat, idx3)
    bias_g, = _make_sc_bias_gather((F * V) // D, D, B * F, NW, CH)(
        bias16, bidx3, lidx2)
    embs = rows.reshape(B, F * D)
    biases = bias_g.reshape(B, F)

    n0 = jnp.repeat(nas_logits[:, 0], D).reshape(1, F * D)
    n1 = jnp.repeat(nas_logits[:, 1], D).reshape(1, F * D)

    scale, shift = pl.pallas_call(
        _stats_body,
        out_shape=[jax.ShapeDtypeStruct((1, F * D), jnp.float32)] * 2,
    )(embs, n0, n1)

    BM = 512
    NB = B // BM

    def cmap(i):
        return (0, 0)

    out = pl.pallas_call(
        _mlp_body,
        grid=(NB,),
        in_specs=[
            pl.BlockSpec((BM, F * D), lambda i: (i, 0)),
            pl.BlockSpec((BM, F), lambda i: (i, 0)),
            pl.BlockSpec((1, F * D), cmap),
            pl.BlockSpec((1, F * D), cmap),
            pl.BlockSpec(W1.shape, cmap),
            pl.BlockSpec((1, W1.shape[1]), cmap),
            pl.BlockSpec(W2.shape, cmap),
            pl.BlockSpec((1, W2.shape[1]), cmap),
            pl.BlockSpec(W3.shape, cmap),
            pl.BlockSpec((1, W3.shape[1]), cmap),
            pl.BlockSpec(W4.shape, cmap),
            pl.BlockSpec((1, W4.shape[1]), cmap),
        ],
        out_specs=pl.BlockSpec((BM, 1), lambda i: (i, 0)),
        out_shape=jax.ShapeDtypeStruct((B, 1), jnp.float32),
    )(embs, biases, scale, shift,
      W1, b1.reshape(1, -1), W2, b2.reshape(1, -1),
      W3, b3.reshape(1, -1), W4, b4.reshape(1, -1))

    return out.reshape(B)
